# baseline (speedup 1.0000x reference)
import jax
import jax.numpy as jnp
from jax.experimental import pallas as pl
from jax.experimental.pallas import tpu as pltpu
from jax.experimental.pallas import tpu_sc as plsc

_SEQ = 4096
_REP = 8
_D = 128
_ROWS = _SEQ // _REP
_NCHUNKS = 8


def _sc_body(table_hbm, out_hbm, stage_v, sem_in, sem_out):
    csize = _ROWS // _NCHUNKS
    stages = []
    for k in range(_NCHUNKS):
        rows = pl.ds(k * csize, csize)
        stages.append(
            [
                pltpu.async_copy(
                    table_hbm.at[rows],
                    stage_v.at[rows, pl.ds(r, 1)],
                    sem_in,
                )
                for r in range(_REP)
            ]
        )
    writes = []
    for k in range(_NCHUNKS):
        rows = pl.ds(k * csize, csize)
        for c in stages[k]:
            c.wait()
        writes.append(
            pltpu.async_copy(stage_v.at[rows], out_hbm.at[rows], sem_out)
        )
    for c in writes:
        c.wait()


@jax.jit
def _position_encode(table):
    mesh = plsc.ScalarSubcoreMesh(axis_name="c", num_cores=1)
    out = pl.kernel(
        _sc_body,
        out_type=jax.ShapeDtypeStruct((_ROWS, _REP, _D), jnp.float32),
        mesh=mesh,
        scratch_types=[
            pltpu.VMEM_SHARED((_ROWS, _REP, _D), jnp.float32),
            pltpu.SemaphoreType.DMA,
            pltpu.SemaphoreType.DMA,
        ],
    )(table.reshape(table.shape[0], 1, _D))
    return out.reshape(_SEQ, _D)


def kernel(x, E_absolute_position):
    del x
    return _position_encode(E_absolute_position)


# speedup vs baseline: 1.0768x; 1.0768x over previous
import jax
import jax.numpy as jnp
from jax.experimental import pallas as pl
from jax.experimental.pallas import tpu as pltpu
from jax.experimental.pallas import tpu_sc as plsc

_SEQ = 4096
_REP = 8
_D = 128
_ROWS = _SEQ // _REP


def _sc_body(table_hbm, out_hbm, stage_v, sem):
    pltpu.sync_copy(table_hbm.at[pl.ds(0, _ROWS)], stage_v)
    writes = [
        pltpu.async_copy(stage_v, out_hbm.at[:, pl.ds(r, 1)], sem)
        for r in range(_REP)
    ]
    for c in writes:
        c.wait()


@jax.jit
def _position_encode(table):
    mesh = plsc.ScalarSubcoreMesh(axis_name="c", num_cores=1)
    out = pl.kernel(
        _sc_body,
        out_type=jax.ShapeDtypeStruct((_ROWS, _REP, _D), jnp.float32),
        mesh=mesh,
        scratch_types=[
            pltpu.VMEM_SHARED((_ROWS, 1, _D), jnp.float32),
            pltpu.SemaphoreType.DMA,
        ],
    )(table.reshape(table.shape[0], 1, _D))
    return out.reshape(_SEQ, _D)


def kernel(x, E_absolute_position):
    del x
    return _position_encode(E_absolute_position)
